# baseline (speedup 1.0000x reference)
import functools

import jax
import jax.numpy as jnp
from jax import lax
from jax.experimental import pallas as pl
from jax.experimental.pallas import tpu as pltpu
from jax.experimental.pallas import tpu_sc as plsc

RANK = 32
LANES = 16
CHUNK = 128
PACK = 128 // RANK


def _mf_body(u_hbm, i_hbm, mu_hbm, bu_hbm, bi_hbm, U_hbm, V_hbm, out_hbm,
             uo, io, u4, i4, ubuf, vbuf, buv, biv, muv, outv, sem,
             *, bpw, nch, nc):
  c = lax.axis_index("c")
  s = lax.axis_index("s")
  wid = s * nc + c
  base = wid * bpw
  gpc = CHUNK // LANES

  for j in range(nch):
    pltpu.sync_copy(u_hbm.at[pl.ds(base + j * CHUNK, CHUNK)], uo.at[j])
    pltpu.sync_copy(i_hbm.at[pl.ds(base + j * CHUNK, CHUNK)], io.at[j])
  pltpu.sync_copy(mu_hbm, muv)

  for j in range(nch):
    for t in range(gpc):
      sl = pl.ds(t * LANES, LANES)
      u4[j, sl] = lax.shift_right_logical(uo[j, sl], 2)
      i4[j, sl] = lax.shift_right_logical(io[j, sl], 2)

  bias_copies = []
  for j in range(nch):
    sl = pl.ds(j * CHUNK, CHUNK)
    bias_copies.append(pltpu.async_copy(bu_hbm.at[uo.at[j]], buv.at[sl], sem))
    bias_copies.append(pltpu.async_copy(bi_hbm.at[io.at[j]], biv.at[sl], sem))
  for cp in bias_copies:
    cp.wait()

  lane = lax.iota(jnp.int32, LANES)
  mu_vec = muv[...]

  for j in range(nch):
    cu = pltpu.async_copy(U_hbm.at[u4.at[j]], ubuf, sem)
    cv = pltpu.async_copy(V_hbm.at[i4.at[j]], vbuf, sem)
    cu.wait()
    cv.wait()

    def grp_body(g, carry, j=j):
      gsl = pl.ds(g * LANES, LANES)
      off_u = lax.shift_left(jnp.bitwise_and(uo[j, gsl], 3), 5)
      off_i = lax.shift_left(jnp.bitwise_and(io[j, gsl], 3), 5)
      rows = g * LANES + lane
      csl = pl.ds(j * CHUNK + g * LANES, LANES)
      acc = buv[csl] + biv[csl] + mu_vec
      for r in range(RANK):
        gu = plsc.load_gather(ubuf, [rows, off_u + r])
        gv = plsc.load_gather(vbuf, [rows, off_i + r])
        acc = acc + gu * gv
      outv[csl] = acc
      return carry

    lax.fori_loop(0, gpc, grp_body, 0)

  pltpu.sync_copy(outv, out_hbm.at[pl.ds(base, bpw)])


def kernel(u, i, mu, bu, bi, U, V):
  batch = u.shape[0]
  info = plsc.get_sparse_core_info()
  nc, ns = info.num_cores, info.num_subcores
  nw = nc * ns
  bpw = batch // nw
  nch = bpw // CHUNK

  mu_vec = jnp.broadcast_to(mu, (LANES,)).astype(jnp.float32)
  bu_flat = bu.reshape(-1)
  bi_flat = bi.reshape(-1)
  U_packed = U.reshape(U.shape[0] // PACK, PACK * RANK)
  V_packed = V.reshape(V.shape[0] // PACK, PACK * RANK)

  mesh = plsc.VectorSubcoreMesh(core_axis_name="c", subcore_axis_name="s")
  body = functools.partial(_mf_body, bpw=bpw, nch=nch, nc=nc)
  fn = pl.kernel(
      body,
      mesh=mesh,
      compiler_params=pltpu.CompilerParams(
          needs_layout_passes=False, use_tc_tiling_on_sc=False),
      out_type=jax.ShapeDtypeStruct((batch,), jnp.float32),
      scratch_types=[
          pltpu.VMEM((nch, CHUNK), jnp.int32),
          pltpu.VMEM((nch, CHUNK), jnp.int32),
          pltpu.VMEM((nch, CHUNK), jnp.int32),
          pltpu.VMEM((nch, CHUNK), jnp.int32),
          pltpu.VMEM((CHUNK, 128), jnp.float32),
          pltpu.VMEM((CHUNK, 128), jnp.float32),
          pltpu.VMEM((bpw,), jnp.float32),
          pltpu.VMEM((bpw,), jnp.float32),
          pltpu.VMEM((LANES,), jnp.float32),
          pltpu.VMEM((bpw,), jnp.float32),
          pltpu.SemaphoreType.DMA,
      ],
  )
  return fn(u, i, mu_vec, bu_flat, bi_flat, U_packed, V_packed)


# speedup vs baseline: 1.0171x; 1.0171x over previous
import functools

import jax
import jax.numpy as jnp
from jax import lax
from jax.experimental import pallas as pl
from jax.experimental.pallas import tpu as pltpu
from jax.experimental.pallas import tpu_sc as plsc

RANK = 32
LANES = 16
CHUNK = 128


def _mf_body(u_hbm, i_hbm, mu_hbm, bu_hbm, bi_hbm, U_hbm, V_hbm, out_hbm,
             uidx, iidx, urows, vrows, buv, biv, muv, st, outv, sem,
             *, bpw, nch, nc):
  c = lax.axis_index("c")
  s = lax.axis_index("s")
  wid = s * nc + c
  base = wid * bpw

  for j in range(nch):
    pltpu.sync_copy(u_hbm.at[pl.ds(base + j * CHUNK, CHUNK)], uidx.at[j])
    pltpu.sync_copy(i_hbm.at[pl.ds(base + j * CHUNK, CHUNK)], iidx.at[j])
  pltpu.sync_copy(mu_hbm, muv)

  copies = []
  for j in range(nch):
    sl = pl.ds(j * CHUNK, CHUNK)
    copies.append(pltpu.async_copy(U_hbm.at[uidx.at[j]], urows.at[sl], sem))
    copies.append(pltpu.async_copy(V_hbm.at[iidx.at[j]], vrows.at[sl], sem))
    copies.append(pltpu.async_copy(bu_hbm.at[uidx.at[j]], buv.at[sl], sem))
    copies.append(pltpu.async_copy(bi_hbm.at[iidx.at[j]], biv.at[sl], sem))
  for cp in copies:
    cp.wait()

  lane = lax.iota(jnp.int32, LANES)

  def dot_body(b, carry):
    u0 = urows[b, pl.ds(0, LANES)]
    u1 = urows[b, pl.ds(LANES, LANES)]
    v0 = vrows[b, pl.ds(0, LANES)]
    v1 = vrows[b, pl.ds(LANES, LANES)]
    part = u0 * v0 + u1 * v1
    plsc.store_scatter(st, [lane * bpw + b], part)
    return carry

  lax.fori_loop(0, bpw, dot_body, 0)

  mu_vec = muv[...]

  def red_body(g, carry):
    sl = pl.ds(g * LANES, LANES)
    acc = buv[sl] + biv[sl] + mu_vec
    for k in range(LANES):
      acc = acc + st[pl.ds(k * bpw + g * LANES, LANES)]
    outv[sl] = acc
    return carry

  lax.fori_loop(0, bpw // LANES, red_body, 0)

  pltpu.sync_copy(outv, out_hbm.at[pl.ds(base, bpw)])


def kernel(u, i, mu, bu, bi, U, V):
  batch = u.shape[0]
  info = plsc.get_sparse_core_info()
  nc, ns = info.num_cores, info.num_subcores
  nw = nc * ns
  bpw = batch // nw
  nch = bpw // CHUNK

  mu_vec = jnp.broadcast_to(mu, (LANES,)).astype(jnp.float32)
  bu_flat = bu.reshape(-1)
  bi_flat = bi.reshape(-1)

  mesh = plsc.VectorSubcoreMesh(core_axis_name="c", subcore_axis_name="s")
  body = functools.partial(_mf_body, bpw=bpw, nch=nch, nc=nc)
  fn = pl.kernel(
      body,
      mesh=mesh,
      compiler_params=pltpu.CompilerParams(
          needs_layout_passes=False, use_tc_tiling_on_sc=False),
      out_type=jax.ShapeDtypeStruct((batch,), jnp.float32),
      scratch_types=[
          pltpu.VMEM((nch, CHUNK), jnp.int32),
          pltpu.VMEM((nch, CHUNK), jnp.int32),
          pltpu.VMEM((bpw, RANK), jnp.float32),
          pltpu.VMEM((bpw, RANK), jnp.float32),
          pltpu.VMEM((bpw,), jnp.float32),
          pltpu.VMEM((bpw,), jnp.float32),
          pltpu.VMEM((LANES,), jnp.float32),
          pltpu.VMEM((LANES * bpw,), jnp.float32),
          pltpu.VMEM((bpw,), jnp.float32),
          pltpu.SemaphoreType.DMA,
      ],
  )
  return fn(u, i, mu_vec, bu_flat, bi_flat, U, V)
